# Initial kernel scaffold; baseline (speedup 1.0000x reference)
import jax
import jax.numpy as jnp
from jax.experimental import pallas as pl


def kernel(xyz, features, W0_0, W0_1, W0_2, W1_0, W1_1, W1_2):
    raise NotImplementedError("write your pallas kernel here")


# speedup vs baseline: 1.0002x; 1.0002x over previous
import jax
import jax.numpy as jnp

_NPOINT = 512
_NSAMPLES = [16, 32]


def _index_points(points, idx):
    return jax.vmap(lambda p, i: p[i])(points, idx)


def _fps(xyz, npoint):
    B, N, _ = xyz.shape
    key = jax.random.key(42)
    farthest0 = jax.random.randint(key, (B,), 0, N)
    distance0 = jnp.full((B, N), 1e10, dtype=xyz.dtype)
    batch_idx = jnp.arange(B)

    def step(carry, _):
        distance, farthest = carry
        centroid = xyz[batch_idx, farthest][:, None, :]
        dist = jnp.sum((xyz - centroid) ** 2, axis=-1)
        distance = jnp.minimum(distance, dist)
        new_farthest = jnp.argmax(distance, axis=1)
        return (distance, new_farthest), farthest

    _, cent = jax.lax.scan(step, (distance0, farthest0), None, length=npoint)
    return cent.T


def _mlp(x, Ws):
    for W in Ws:
        x = jnp.einsum('bskc,oc->bsko', x, W)
        mean = jnp.mean(x, axis=(0, 1, 2), keepdims=True)
        var = jnp.var(x, axis=(0, 1, 2), keepdims=True)
        x = (x - mean) / jnp.sqrt(var + 1e-5)
        x = jax.nn.relu(x)
    return x


def kernel(xyz, features, W0_0, W0_1, W0_2, W1_0, W1_1, W1_2):
    Ws_list = [[W0_0, W0_1, W0_2], [W1_0, W1_1, W1_2]]
    fps_idx = _fps(jax.lax.stop_gradient(xyz), _NPOINT)
    new_xyz = _index_points(xyz, fps_idx)
    outs = []
    for nsample, Ws in zip(_NSAMPLES, Ws_list):
        diff = new_xyz[:, :, None, :] - xyz[:, None, :, :]
        dists = jnp.sqrt(jnp.sum(diff * diff, axis=-1))
        _, idx = jax.lax.top_k(jax.lax.stop_gradient(-dists), nsample)
        grouped_xyz = _index_points(xyz, idx) - new_xyz[:, :, None, :]
        grouped_feat = _index_points(features, idx)
        grouped = jnp.concatenate([grouped_xyz, grouped_feat], axis=-1)
        enc = _mlp(grouped, Ws)
        outs.append(jnp.max(enc, axis=2))
    fused = jnp.concatenate(outs, axis=-1)
    return new_xyz, fused


# speedup vs baseline: 13.7446x; 13.7416x over previous
import functools

import jax
import jax.numpy as jnp
from jax import lax
from jax.experimental import pallas as pl
from jax.experimental.pallas import tpu as pltpu
from jax.experimental.pallas import tpu_sc as plsc

_NPOINT = 512
_NSAMPLES = [16, 32]
_KMAX = 32
_CPAD = 128
_EPS = 1e-5


def _fps_body(px_ref, py_ref, pz_ref, f0_ref, ox_ref, oy_ref, oz_ref):
    px = px_ref[...]
    py = py_ref[...]
    pz = pz_ref[...]
    B, N = px.shape
    S = ox_ref.shape[1]
    lane = lax.broadcasted_iota(jnp.int32, (B, N), 1)
    col = lax.broadcasted_iota(jnp.int32, (B, S), 1)

    def step(i, carry):
        dist_min, f, ax, ay, az = carry
        onehot = lane == f
        cx = jnp.sum(jnp.where(onehot, px, 0.0), axis=1, keepdims=True)
        cy = jnp.sum(jnp.where(onehot, py, 0.0), axis=1, keepdims=True)
        cz = jnp.sum(jnp.where(onehot, pz, 0.0), axis=1, keepdims=True)
        ax = jnp.where(col == i, cx, ax)
        ay = jnp.where(col == i, cy, ay)
        az = jnp.where(col == i, cz, az)
        dx = px - cx
        dy = py - cy
        dz = pz - cz
        d = dx * dx + dy * dy + dz * dz
        dist_min = jnp.minimum(dist_min, d)
        f = jnp.argmax(dist_min, axis=1).astype(jnp.int32)[:, None]
        return dist_min, f, ax, ay, az

    dist0 = jnp.full((B, N), 1e10, dtype=jnp.float32)
    zeros = jnp.zeros((B, S), dtype=jnp.float32)
    f0 = f0_ref[...]
    _, _, ax, ay, az = lax.fori_loop(0, S, step, (dist0, f0, zeros, zeros, zeros))
    ox_ref[...] = ax
    oy_ref[...] = ay
    oz_ref[...] = az


def _fps(px, py, pz, f0):
    B, N = px.shape
    out = jax.ShapeDtypeStruct((B, _NPOINT), jnp.float32)
    return pl.pallas_call(
        _fps_body,
        out_shape=(out, out, out),
    )(px, py, pz, f0)


_QBLK = 128


def _knn_body(px_ref, py_ref, pz_ref, qx_ref, qy_ref, qz_ref, idx_ref):
    b = pl.program_id(0)
    px = px_ref[0, 0, :][None, :]
    py = py_ref[0, 0, :][None, :]
    pz = pz_ref[0, 0, :][None, :]
    qx = qx_ref[0, 0, :][:, None]
    qy = qy_ref[0, 0, :][:, None]
    qz = qz_ref[0, 0, :][:, None]
    dx = qx - px
    dy = qy - py
    dz = qz - pz
    d2 = dx * dx + dy * dy + dz * dz
    N = d2.shape[1]
    lane = lax.broadcasted_iota(jnp.int32, (_QBLK, N), 1)
    base = b * N
    for k in range(_KMAX):
        am = jnp.argmin(d2, axis=1).astype(jnp.int32)
        idx_ref[0, pl.ds(k, 1), :] = (am + base)[None, :]
        d2 = jnp.where(lane == am[:, None], jnp.inf, d2)


def _knn(px, py, pz, nx, ny, nz):
    B, N = px.shape
    S = nx.shape[1]
    grid = (B, S // _QBLK)
    p_spec = pl.BlockSpec((1, 1, N), lambda b, q: (b, 0, 0))
    q_spec = pl.BlockSpec((1, 1, _QBLK), lambda b, q: (b, 0, q))
    idx_spec = pl.BlockSpec((1, _KMAX, _QBLK), lambda b, q: (b, 0, q))
    return pl.pallas_call(
        _knn_body,
        grid=grid,
        in_specs=[p_spec, p_spec, p_spec, q_spec, q_spec, q_spec],
        out_specs=idx_spec,
        out_shape=jax.ShapeDtypeStruct((B, _KMAX, S), jnp.int32),
    )(px[:, None, :], py[:, None, :], pz[:, None, :],
      nx[:, None, :], ny[:, None, :], nz[:, None, :])


_GCHUNK = 512


def _gather_rows(table, idx):
    R = idx.shape[0]
    NW = 32
    per_w = R // NW
    n_chunks = per_w // _GCHUNK
    mesh = plsc.VectorSubcoreMesh(core_axis_name="c", subcore_axis_name="s")

    @functools.partial(
        pl.kernel,
        out_type=jax.ShapeDtypeStruct((R, _CPAD), jnp.float32),
        mesh=mesh,
        scratch_types=[
            pltpu.VMEM((_GCHUNK,), jnp.int32),
            pltpu.VMEM((_GCHUNK, _CPAD), jnp.float32),
            pltpu.SemaphoreType.DMA,
        ],
    )
    def gather_kernel(table_hbm, idx_hbm, out_hbm, idx_v, rows_v, sem):
        wid = lax.axis_index("s") * 2 + lax.axis_index("c")
        base = wid * per_w
        for c in range(n_chunks):
            off = base + c * _GCHUNK
            pltpu.sync_copy(idx_hbm.at[pl.ds(off, _GCHUNK)], idx_v)
            pltpu.async_copy(table_hbm.at[idx_v], rows_v, sem).wait()
            pltpu.sync_copy(rows_v, out_hbm.at[pl.ds(off, _GCHUNK)])

    return gather_kernel(table, idx)


_GBLK = 256


def _mlp_l0_body(K, g_ref, nxyz_ref, w_ref, w3_ref, y_ref, st_ref):
    gb = pl.program_id(0)
    g = g_ref[...]
    g = g[:, :K, :].reshape(_GBLK * K, _CPAD)
    y = jnp.dot(g, w_ref[...], preferred_element_type=jnp.float32)
    nxyz = nxyz_ref[...]
    w3 = w3_ref[...]
    cy = (
        nxyz[:, 0:1] * w3[0:1, :]
        + nxyz[:, 1:2] * w3[1:2, :]
        + nxyz[:, 2:3] * w3[2:3, :]
    )
    y = (y.reshape(_GBLK, K, -1) - cy[:, None, :]).reshape(_GBLK * K, -1)
    s = jnp.sum(y, axis=0, keepdims=True)
    ss = jnp.sum(y * y, axis=0, keepdims=True)
    st = jnp.concatenate([s, ss], axis=0)

    @pl.when(gb == 0)
    def _():
        st_ref[...] = jnp.zeros_like(st_ref)

    st_ref[...] += st
    y_ref[...] = y


def _mlp_l0(g4, nxyz, w, w3, K):
    G = g4.shape[0]
    Cout = w.shape[1]
    grid = (G // _GBLK,)
    return pl.pallas_call(
        functools.partial(_mlp_l0_body, K),
        grid=grid,
        in_specs=[
            pl.BlockSpec((_GBLK, _KMAX, _CPAD), lambda i: (i, 0, 0)),
            pl.BlockSpec((_GBLK, 3), lambda i: (i, 0)),
            pl.BlockSpec(w.shape, lambda i: (0, 0)),
            pl.BlockSpec(w3.shape, lambda i: (0, 0)),
        ],
        out_specs=[
            pl.BlockSpec((_GBLK * K, Cout), lambda i: (i, 0)),
            pl.BlockSpec((2, Cout), lambda i: (0, 0)),
        ],
        out_shape=[
            jax.ShapeDtypeStruct((G * K, Cout), jnp.float32),
            jax.ShapeDtypeStruct((2, Cout), jnp.float32),
        ],
    )(g4, nxyz, w, w3)


def _norm_relu(y, st, r):
    mu = st[0:1, :] / r
    var = st[1:2, :] / r - mu * mu
    inv = lax.rsqrt(var + _EPS)
    return jnp.maximum((y - mu) * inv, 0.0)


def _mlp_mid_body(r, y_ref, st_ref, w_ref, o_ref, ost_ref):
    gb = pl.program_id(0)
    x = _norm_relu(y_ref[...], st_ref[...], r)
    y = jnp.dot(x, w_ref[...], preferred_element_type=jnp.float32)
    s = jnp.sum(y, axis=0, keepdims=True)
    ss = jnp.sum(y * y, axis=0, keepdims=True)
    st = jnp.concatenate([s, ss], axis=0)

    @pl.when(gb == 0)
    def _():
        ost_ref[...] = jnp.zeros_like(ost_ref)

    ost_ref[...] += st
    o_ref[...] = y


def _mlp_mid(y, st, w, K):
    R, Cin = y.shape
    Cout = w.shape[1]
    rb = _GBLK * K
    grid = (R // rb,)
    return pl.pallas_call(
        functools.partial(_mlp_mid_body, float(R)),
        grid=grid,
        in_specs=[
            pl.BlockSpec((rb, Cin), lambda i: (i, 0)),
            pl.BlockSpec((2, Cin), lambda i: (0, 0)),
            pl.BlockSpec(w.shape, lambda i: (0, 0)),
        ],
        out_specs=[
            pl.BlockSpec((rb, Cout), lambda i: (i, 0)),
            pl.BlockSpec((2, Cout), lambda i: (0, 0)),
        ],
        out_shape=[
            jax.ShapeDtypeStruct((R, Cout), jnp.float32),
            jax.ShapeDtypeStruct((2, Cout), jnp.float32),
        ],
    )(y, st, w)


def _mlp_final_body(r, K, y_ref, st_ref, o_ref):
    x = _norm_relu(y_ref[...], st_ref[...], r)
    C = x.shape[1]
    o_ref[...] = jnp.max(x.reshape(_GBLK, K, C), axis=1)


def _mlp_final(y, st, K):
    R, C = y.shape
    rb = _GBLK * K
    grid = (R // rb,)
    return pl.pallas_call(
        functools.partial(_mlp_final_body, float(R), K),
        grid=grid,
        in_specs=[
            pl.BlockSpec((rb, C), lambda i: (i, 0)),
            pl.BlockSpec((2, C), lambda i: (0, 0)),
        ],
        out_specs=pl.BlockSpec((_GBLK, C), lambda i: (i, 0)),
        out_shape=jax.ShapeDtypeStruct((R // K, C), jnp.float32),
    )(y, st)


def kernel(xyz, features, W0_0, W0_1, W0_2, W1_0, W1_1, W1_2):
    B, N, _ = xyz.shape
    C = features.shape[2]
    S = _NPOINT

    px = xyz[:, :, 0]
    py = xyz[:, :, 1]
    pz = xyz[:, :, 2]
    f0 = jax.random.randint(jax.random.key(42), (B,), 0, N).astype(jnp.int32)

    nx, ny, nz = _fps(px, py, pz, f0[:, None])
    idx = _knn(px, py, pz, nx, ny, nz)
    idx_flat = idx.transpose(0, 2, 1).reshape(-1)

    table = jnp.concatenate(
        [xyz, features, jnp.zeros((B, N, _CPAD - 3 - C), jnp.float32)], axis=-1
    ).reshape(B * N, _CPAD)
    g = _gather_rows(table, idx_flat)
    g4 = g.reshape(B * S, _KMAX, _CPAD)
    nxyz = jnp.stack([nx, ny, nz], axis=-1).reshape(B * S, 3)

    outs = []
    for K, Ws in zip(_NSAMPLES, [[W0_0, W0_1, W0_2], [W1_0, W1_1, W1_2]]):
        w0 = jnp.pad(Ws[0].T, ((0, _CPAD - Ws[0].shape[1]), (0, 0)))
        w3 = Ws[0].T[:3, :]
        y, st = _mlp_l0(g4, nxyz, w0, w3, K)
        y, st = _mlp_mid(y, st, Ws[1].T, K)
        y, st = _mlp_mid(y, st, Ws[2].T, K)
        out = _mlp_final(y, st, K)
        outs.append(out.reshape(B, S, -1))

    new_xyz = jnp.stack([nx, ny, nz], axis=-1)
    fused = jnp.concatenate(outs, axis=-1)
    return new_xyz, fused


# speedup vs baseline: 14.1542x; 1.0298x over previous
import functools

import jax
import jax.numpy as jnp
from jax import lax
from jax.experimental import pallas as pl
from jax.experimental.pallas import tpu as pltpu
from jax.experimental.pallas import tpu_sc as plsc

_NPOINT = 512
_NSAMPLES = [16, 32]
_KMAX = 32
_CPAD = 128
_EPS = 1e-5


def _fps_body(px_ref, py_ref, pz_ref, f0_ref, ox_ref, oy_ref, oz_ref):
    px = px_ref[...]
    py = py_ref[...]
    pz = pz_ref[...]
    B, N = px.shape
    S = ox_ref.shape[1]
    lane = lax.broadcasted_iota(jnp.int32, (B, N), 1)
    col = lax.broadcasted_iota(jnp.int32, (B, S), 1)

    def step(i, carry):
        dist_min, f, ax, ay, az = carry
        onehot = lane == f
        cx = jnp.sum(jnp.where(onehot, px, 0.0), axis=1, keepdims=True)
        cy = jnp.sum(jnp.where(onehot, py, 0.0), axis=1, keepdims=True)
        cz = jnp.sum(jnp.where(onehot, pz, 0.0), axis=1, keepdims=True)
        ax = jnp.where(col == i, cx, ax)
        ay = jnp.where(col == i, cy, ay)
        az = jnp.where(col == i, cz, az)
        dx = px - cx
        dy = py - cy
        dz = pz - cz
        d = dx * dx + dy * dy + dz * dz
        dist_min = jnp.minimum(dist_min, d)
        f = jnp.argmax(dist_min, axis=1).astype(jnp.int32)[:, None]
        return dist_min, f, ax, ay, az

    dist0 = jnp.full((B, N), 1e10, dtype=jnp.float32)
    zeros = jnp.zeros((B, S), dtype=jnp.float32)
    f0 = f0_ref[...]
    _, _, ax, ay, az = lax.fori_loop(0, S, step, (dist0, f0, zeros, zeros, zeros))
    ox_ref[...] = ax
    oy_ref[...] = ay
    oz_ref[...] = az


def _fps(px, py, pz, f0):
    B, N = px.shape
    out = jax.ShapeDtypeStruct((B, _NPOINT), jnp.float32)
    return pl.pallas_call(
        _fps_body,
        out_shape=(out, out, out),
    )(px, py, pz, f0)


_QBLK = 128


def _knn_body(px_ref, py_ref, pz_ref, qx_ref, qy_ref, qz_ref, idx_ref):
    b = pl.program_id(0)
    px = px_ref[0, 0, :][None, :]
    py = py_ref[0, 0, :][None, :]
    pz = pz_ref[0, 0, :][None, :]
    qx = qx_ref[0, 0, :][:, None]
    qy = qy_ref[0, 0, :][:, None]
    qz = qz_ref[0, 0, :][:, None]
    dx = qx - px
    dy = qy - py
    dz = qz - pz
    d2 = dx * dx + dy * dy + dz * dz
    N = d2.shape[1]
    lane = lax.broadcasted_iota(jnp.int32, (_QBLK, N), 1)
    base = b * N
    for k in range(_KMAX):
        am = jnp.argmin(d2, axis=1).astype(jnp.int32)
        idx_ref[0, pl.ds(k, 1), :] = (am + base)[None, :]
        d2 = jnp.where(lane == am[:, None], jnp.inf, d2)


def _knn(px, py, pz, nx, ny, nz):
    B, N = px.shape
    S = nx.shape[1]
    grid = (B, S // _QBLK)
    p_spec = pl.BlockSpec((1, 1, N), lambda b, q: (b, 0, 0))
    q_spec = pl.BlockSpec((1, 1, _QBLK), lambda b, q: (b, 0, q))
    idx_spec = pl.BlockSpec((1, _KMAX, _QBLK), lambda b, q: (b, 0, q))
    return pl.pallas_call(
        _knn_body,
        grid=grid,
        in_specs=[p_spec, p_spec, p_spec, q_spec, q_spec, q_spec],
        out_specs=idx_spec,
        out_shape=jax.ShapeDtypeStruct((B, _KMAX, S), jnp.int32),
    )(px[:, None, :], py[:, None, :], pz[:, None, :],
      nx[:, None, :], ny[:, None, :], nz[:, None, :])


_GCHUNK = 512


def _gather_rows(table, idx):
    R = idx.shape[0]
    NW = 32
    per_w = R // NW
    n_chunks = per_w // _GCHUNK
    mesh = plsc.VectorSubcoreMesh(core_axis_name="c", subcore_axis_name="s")

    @functools.partial(
        pl.kernel,
        out_type=jax.ShapeDtypeStruct((R, _CPAD), jnp.float32),
        mesh=mesh,
        scratch_types=[
            pltpu.VMEM((_GCHUNK,), jnp.int32),
            pltpu.VMEM((_GCHUNK, _CPAD), jnp.float32),
            pltpu.SemaphoreType.DMA,
        ],
    )
    def gather_kernel(table_hbm, idx_hbm, out_hbm, idx_v, rows_v, sem):
        wid = lax.axis_index("s") * 2 + lax.axis_index("c")
        base = wid * per_w
        for c in range(n_chunks):
            off = base + c * _GCHUNK
            pltpu.sync_copy(idx_hbm.at[pl.ds(off, _GCHUNK)], idx_v)
            pltpu.async_copy(table_hbm.at[idx_v], rows_v, sem).wait()
            pltpu.sync_copy(rows_v, out_hbm.at[pl.ds(off, _GCHUNK)])

    return gather_kernel(table, idx)


_GBLK = 256


_PBLK = 2048


def _proj_body(x3_ref, feat_ref, wf_ref, w3_ref, o_ref):
    y = jnp.dot(feat_ref[...], wf_ref[...], preferred_element_type=jnp.float32)
    x3 = x3_ref[...]
    w3 = w3_ref[...]
    o_ref[...] = (
        y
        + x3[:, 0:1] * w3[0:1, :]
        + x3[:, 1:2] * w3[1:2, :]
        + x3[:, 2:3] * w3[2:3, :]
    )


def _proj(x3, feat, wf, w3):
    V = x3.shape[0]
    grid = (V // _PBLK,)
    return pl.pallas_call(
        _proj_body,
        grid=grid,
        in_specs=[
            pl.BlockSpec((_PBLK, 3), lambda i: (i, 0)),
            pl.BlockSpec((_PBLK, feat.shape[1]), lambda i: (i, 0)),
            pl.BlockSpec(wf.shape, lambda i: (0, 0)),
            pl.BlockSpec(w3.shape, lambda i: (0, 0)),
        ],
        out_specs=pl.BlockSpec((_PBLK, _CPAD), lambda i: (i, 0)),
        out_shape=jax.ShapeDtypeStruct((V, _CPAD), jnp.float32),
    )(x3, feat, wf, w3)


def _stats(y):
    s = jnp.sum(y, axis=0, keepdims=True)
    ss = jnp.sum(y * y, axis=0, keepdims=True)
    return jnp.concatenate([s, ss], axis=0)


def _mlp_l0_body(K0, g_ref, nxyz_ref, w3_ref, y0_ref, y1_ref, st0_ref, st1_ref):
    gb = pl.program_id(0)
    g = g_ref[...]
    nxyz = nxyz_ref[...]
    w3 = w3_ref[...]
    cp = (
        nxyz[:, 0:1] * w3[0:1, :]
        + nxyz[:, 1:2] * w3[1:2, :]
        + nxyz[:, 2:3] * w3[2:3, :]
    )
    y = g - cp[:, None, :]
    y0 = y[:, :K0, 0:64].reshape(_GBLK * K0, 64)
    y1 = y[:, :, 64:128].reshape(_GBLK * _KMAX, 64)

    @pl.when(gb == 0)
    def _():
        st0_ref[...] = jnp.zeros_like(st0_ref)
        st1_ref[...] = jnp.zeros_like(st1_ref)

    st0_ref[...] += _stats(y0)
    st1_ref[...] += _stats(y1)
    y0_ref[...] = y0
    y1_ref[...] = y1


def _mlp_l0(g4, nxyz, w3, K0):
    G = g4.shape[0]
    grid = (G // _GBLK,)
    st_shape = jax.ShapeDtypeStruct((2, 64), jnp.float32)
    return pl.pallas_call(
        functools.partial(_mlp_l0_body, K0),
        grid=grid,
        in_specs=[
            pl.BlockSpec((_GBLK, _KMAX, _CPAD), lambda i: (i, 0, 0)),
            pl.BlockSpec((_GBLK, 3), lambda i: (i, 0)),
            pl.BlockSpec(w3.shape, lambda i: (0, 0)),
        ],
        out_specs=[
            pl.BlockSpec((_GBLK * K0, 64), lambda i: (i, 0)),
            pl.BlockSpec((_GBLK * _KMAX, 64), lambda i: (i, 0)),
            pl.BlockSpec((2, 64), lambda i: (0, 0)),
            pl.BlockSpec((2, 64), lambda i: (0, 0)),
        ],
        out_shape=[
            jax.ShapeDtypeStruct((G * K0, 64), jnp.float32),
            jax.ShapeDtypeStruct((G * _KMAX, 64), jnp.float32),
            st_shape,
            st_shape,
        ],
    )(g4, nxyz, w3)


def _norm_relu(y, st, r):
    mu = st[0:1, :] / r
    var = st[1:2, :] / r - mu * mu
    inv = lax.rsqrt(var + _EPS)
    return jnp.maximum((y - mu) * inv, 0.0)


def _mlp_mid_body(r, y_ref, st_ref, w_ref, o_ref, ost_ref):
    gb = pl.program_id(0)
    x = _norm_relu(y_ref[...], st_ref[...], r)
    y = jnp.dot(x, w_ref[...], preferred_element_type=jnp.float32)
    s = jnp.sum(y, axis=0, keepdims=True)
    ss = jnp.sum(y * y, axis=0, keepdims=True)
    st = jnp.concatenate([s, ss], axis=0)

    @pl.when(gb == 0)
    def _():
        ost_ref[...] = jnp.zeros_like(ost_ref)

    ost_ref[...] += st
    o_ref[...] = y


def _mlp_mid(y, st, w, K):
    R, Cin = y.shape
    Cout = w.shape[1]
    rb = _GBLK * K
    grid = (R // rb,)
    return pl.pallas_call(
        functools.partial(_mlp_mid_body, float(R)),
        grid=grid,
        in_specs=[
            pl.BlockSpec((rb, Cin), lambda i: (i, 0)),
            pl.BlockSpec((2, Cin), lambda i: (0, 0)),
            pl.BlockSpec(w.shape, lambda i: (0, 0)),
        ],
        out_specs=[
            pl.BlockSpec((rb, Cout), lambda i: (i, 0)),
            pl.BlockSpec((2, Cout), lambda i: (0, 0)),
        ],
        out_shape=[
            jax.ShapeDtypeStruct((R, Cout), jnp.float32),
            jax.ShapeDtypeStruct((2, Cout), jnp.float32),
        ],
    )(y, st, w)


def _mlp_final_body(r, K, y_ref, st_ref, o_ref):
    x = _norm_relu(y_ref[...], st_ref[...], r)
    C = x.shape[1]
    o_ref[...] = jnp.max(x.reshape(_GBLK, K, C), axis=1)


def _mlp_final(y, st, K):
    R, C = y.shape
    rb = _GBLK * K
    grid = (R // rb,)
    return pl.pallas_call(
        functools.partial(_mlp_final_body, float(R), K),
        grid=grid,
        in_specs=[
            pl.BlockSpec((rb, C), lambda i: (i, 0)),
            pl.BlockSpec((2, C), lambda i: (0, 0)),
        ],
        out_specs=pl.BlockSpec((_GBLK, C), lambda i: (i, 0)),
        out_shape=jax.ShapeDtypeStruct((R // K, C), jnp.float32),
    )(y, st)


def kernel(xyz, features, W0_0, W0_1, W0_2, W1_0, W1_1, W1_2):
    B, N, _ = xyz.shape
    C = features.shape[2]
    S = _NPOINT

    px = xyz[:, :, 0]
    py = xyz[:, :, 1]
    pz = xyz[:, :, 2]
    f0 = jax.random.randint(jax.random.key(42), (B,), 0, N).astype(jnp.int32)

    nx, ny, nz = _fps(px, py, pz, f0[:, None])
    idx = _knn(px, py, pz, nx, ny, nz)
    idx_flat = idx.transpose(0, 2, 1).reshape(-1)

    wf = jnp.concatenate([W0_0.T[3:], W1_0.T[3:]], axis=1)
    w3 = jnp.concatenate([W0_0.T[:3], W1_0.T[:3]], axis=1)
    proj = _proj(xyz.reshape(B * N, 3), features.reshape(B * N, C), wf, w3)
    g = _gather_rows(proj, idx_flat)
    g4 = g.reshape(B * S, _KMAX, _CPAD)
    nxyz = jnp.stack([nx, ny, nz], axis=-1).reshape(B * S, 3)

    y0, y1, st0, st1 = _mlp_l0(g4, nxyz, w3, _NSAMPLES[0])
    outs = []
    for y, st, K, Ws in [
        (y0, st0, _NSAMPLES[0], [W0_1, W0_2]),
        (y1, st1, _NSAMPLES[1], [W1_1, W1_2]),
    ]:
        y, st = _mlp_mid(y, st, Ws[0].T, K)
        y, st = _mlp_mid(y, st, Ws[1].T, K)
        out = _mlp_final(y, st, K)
        outs.append(out.reshape(B, S, -1))

    new_xyz = jnp.stack([nx, ny, nz], axis=-1)
    fused = jnp.concatenate(outs, axis=-1)
    return new_xyz, fused
